# Initial kernel scaffold; baseline (speedup 1.0000x reference)
import jax
import jax.numpy as jnp
from jax.experimental import pallas as pl


def kernel(x, edge_index, edge_weight, W0_1, Wih_1, Whh_1, bih_1, bhh_1, W0_2, Wih_2, Whh_2, bih_2, bhh_2, Wlin, blin):
    raise NotImplementedError("write your pallas kernel here")


# speedup vs baseline: 9.0409x; 9.0409x over previous
import functools

import jax
import jax.numpy as jnp
from jax import lax
from jax.experimental import pallas as pl
from jax.experimental.pallas import tpu as pltpu
from jax.experimental.pallas import tpu_sc as plsc

N = 10000
E = 320000
D = 128
C = 40

NC = 2
NS = 16
NT = NC * NS
EPT = E // NT
K = 80
NCHUNK = EPT // K
SZ = 624
ZR = 208

_mesh = plsc.VectorSubcoreMesh(
    core_axis_name="c", subcore_axis_name="s", num_cores=NC, num_subcores=NS
)
_sc_params = pltpu.CompilerParams(needs_layout_passes=False)

f32 = jnp.float32


@functools.partial(
    pl.kernel,
    out_type=jax.ShapeDtypeStruct((NT, N), f32),
    mesh=_mesh,
    compiler_params=_sc_params,
    scratch_types=[
        pltpu.VMEM((N,), f32),
        pltpu.VMEM((K,), jnp.int32),
        pltpu.VMEM((K,), f32),
    ],
)
def _sc_deg(edge_ref, ew_ref, out_ref, degl, colb, ewb):
    c = lax.axis_index("c")
    s = lax.axis_index("s")
    wid = c * NS + s

    def zero_body(i, _):
        degl[pl.ds(i * 16, 16)] = jnp.zeros((16,), f32)
        return 0

    lax.fori_loop(0, N // 16, zero_body, 0)

    def chunk_body(j, _):
        base = wid * EPT + j * K
        pltpu.sync_copy(edge_ref.at[pl.ds(E + base, K)], colb)
        pltpu.sync_copy(ew_ref.at[pl.ds(base, K)], ewb)

        def vec_body(i, _):
            cv = colb[pl.ds(i * 16, 16)]
            ev = ewb[pl.ds(i * 16, 16)]
            plsc.addupdate_scatter(degl, [cv], ev)
            return 0

        lax.fori_loop(0, K // 16, vec_body, 0)
        return 0

    lax.fori_loop(0, NCHUNK, chunk_body, 0)
    pltpu.sync_copy(degl, out_ref.at[wid])


@functools.partial(
    pl.kernel,
    out_type=jax.ShapeDtypeStruct((NC, N, D), f32),
    mesh=_mesh,
    compiler_params=_sc_params,
    scratch_types=[
        pltpu.VMEM_SHARED((N, D), f32),
        pltpu.VMEM((ZR, D), f32),
        pltpu.VMEM((K,), jnp.int32),
        pltpu.VMEM((K,), jnp.int32),
        pltpu.VMEM((K,), f32),
        pltpu.VMEM((K, D), f32),
    ],
)
def _sc_conv(xw_ref, edge_ref, ew_ref, out_ref, acc, zbuf, rowb, colb, ewb, rows):
    c = lax.axis_index("c")
    s = lax.axis_index("s")
    wid = c * NS + s

    def zrow(r, _):
        for g in range(D // 16):
            zbuf[r, pl.ds(g * 16, 16)] = jnp.zeros((16,), f32)
        return 0

    lax.fori_loop(0, ZR, zrow, 0)

    def zcopy(m, _):
        pltpu.sync_copy(zbuf, acc.at[pl.ds(s * SZ + m * ZR, ZR)])
        return 0

    lax.fori_loop(0, SZ // ZR, zcopy, 0)

    @pl.when(s == NS - 1)
    def _ztail():
        pltpu.sync_copy(zbuf.at[pl.ds(0, 16)], acc.at[pl.ds(NS * SZ, 16)])

    plsc.subcore_barrier()

    def chunk_body(j, _):
        base = wid * EPT + j * K
        pltpu.sync_copy(edge_ref.at[pl.ds(base, K)], rowb)
        pltpu.sync_copy(edge_ref.at[pl.ds(E + base, K)], colb)
        pltpu.sync_copy(ew_ref.at[pl.ds(base, K)], ewb)
        pltpu.sync_copy(xw_ref.at[rowb], rows)

        def edge_body(e, _):
            w = plsc.load_gather(ewb, [jnp.full((16,), e, jnp.int32)])
            for g in range(D // 16):
                rows[e, pl.ds(g * 16, 16)] = rows[e, pl.ds(g * 16, 16)] * w
            return 0

        lax.fori_loop(0, K, edge_body, 0)
        pltpu.sync_copy(rows, acc.at[colb], add=True)
        return 0

    lax.fori_loop(0, NCHUNK, chunk_body, 0)
    plsc.subcore_barrier()

    def outcopy(m, _):
        lo = s * SZ + m * ZR
        pltpu.sync_copy(acc.at[pl.ds(lo, ZR)], out_ref.at[c, pl.ds(lo, ZR)])
        return 0

    lax.fori_loop(0, SZ // ZR, outcopy, 0)

    @pl.when(s == NS - 1)
    def _otail():
        pltpu.sync_copy(acc.at[pl.ds(NS * SZ, 16)],
                        out_ref.at[c, pl.ds(NS * SZ, 16)])


def _gru(W0, WihT, WhhT, bih, bhh):
    gi = jnp.dot(W0, WihT, preferred_element_type=f32) + bih
    gh = jnp.dot(W0, WhhT, preferred_element_type=f32) + bhh
    r = jax.nn.sigmoid(gi[:, :D] + gh[:, :D])
    z = jax.nn.sigmoid(gi[:, D:2 * D] + gh[:, D:2 * D])
    n = jnp.tanh(gi[:, 2 * D:] + r * gh[:, 2 * D:])
    return (1.0 - z) * n + z * W0


def _tc_gru_body(W01, WihT1, WhhT1, bih1, bhh1, W02, WihT2, WhhT2, bih2, bhh2,
                 W1_out, W2_out):
    W1_out[...] = _gru(W01[...], WihT1[...], WhhT1[...], bih1[...], bhh1[...])
    W2_out[...] = _gru(W02[...], WihT2[...], WhhT2[...], bih2[...], bhh2[...])


def _tc_gru(W01, WihT1, WhhT1, bih1, bhh1, W02, WihT2, WhhT2, bih2, bhh2):
    return pl.pallas_call(
        _tc_gru_body,
        out_shape=(jax.ShapeDtypeStruct((D, D), f32),
                   jax.ShapeDtypeStruct((D, D), f32)),
    )(W01, WihT1, WhhT1, bih1, bhh1, W02, WihT2, WhhT2, bih2, bhh2)


_RB = 1000
_NG = N // _RB


def _tc_xw1_body(x_ref, W1_ref, degT_ref, xw_ref, dinv_ref):
    deg = jnp.sum(degT_ref[...], axis=1, keepdims=True) + 1.0
    dinv = lax.rsqrt(deg)
    dinv_ref[...] = dinv
    xw_ref[...] = dinv * jnp.dot(x_ref[...], W1_ref[...],
                                 preferred_element_type=f32)


def _tc_xw1(x, W1, deg_partsT):
    return pl.pallas_call(
        _tc_xw1_body,
        grid=(_NG,),
        in_specs=[
            pl.BlockSpec((_RB, D), lambda i: (i, 0)),
            pl.BlockSpec((D, D), lambda i: (0, 0)),
            pl.BlockSpec((_RB, NT), lambda i: (i, 0)),
        ],
        out_specs=(
            pl.BlockSpec((_RB, D), lambda i: (i, 0)),
            pl.BlockSpec((_RB, 1), lambda i: (i, 0)),
        ),
        out_shape=(jax.ShapeDtypeStruct((N, D), f32),
                   jax.ShapeDtypeStruct((N, 1), f32)),
    )(x, W1, deg_partsT)


def _tc_mid_body(p_ref, xw_ref, dinv_ref, W2_ref, out_ref):
    dinv = dinv_ref[...]
    h = jnp.maximum(dinv * (p_ref[0] + p_ref[1] + xw_ref[...]), 0.0)
    out_ref[...] = dinv * jnp.dot(h, W2_ref[...], preferred_element_type=f32)


def _tc_mid(p, xw1p, dinv, W2):
    return pl.pallas_call(
        _tc_mid_body,
        grid=(_NG,),
        in_specs=[
            pl.BlockSpec((NC, _RB, D), lambda i: (0, i, 0)),
            pl.BlockSpec((_RB, D), lambda i: (i, 0)),
            pl.BlockSpec((_RB, 1), lambda i: (i, 0)),
            pl.BlockSpec((D, D), lambda i: (0, 0)),
        ],
        out_specs=pl.BlockSpec((_RB, D), lambda i: (i, 0)),
        out_shape=jax.ShapeDtypeStruct((N, D), f32),
    )(p, xw1p, dinv, W2)


def _tc_final_body(q_ref, xw_ref, dinv_ref, WlinT_ref, blin_ref, out_ref):
    dinv = dinv_ref[...]
    h = jnp.maximum(dinv * (q_ref[0] + q_ref[1] + xw_ref[...]), 0.0)
    logits = jnp.dot(h, WlinT_ref[...], preferred_element_type=f32) + blin_ref[...]
    m = jnp.max(logits, axis=-1, keepdims=True)
    lse = m + jnp.log(jnp.sum(jnp.exp(logits - m), axis=-1, keepdims=True))
    out_ref[...] = logits - lse


def _tc_final(q, xw2p, dinv, WlinT, blin2):
    return pl.pallas_call(
        _tc_final_body,
        grid=(_NG,),
        in_specs=[
            pl.BlockSpec((NC, _RB, D), lambda i: (0, i, 0)),
            pl.BlockSpec((_RB, D), lambda i: (i, 0)),
            pl.BlockSpec((_RB, 1), lambda i: (i, 0)),
            pl.BlockSpec((D, C), lambda i: (0, 0)),
            pl.BlockSpec((1, C), lambda i: (0, 0)),
        ],
        out_specs=pl.BlockSpec((_RB, C), lambda i: (i, 0)),
        out_shape=jax.ShapeDtypeStruct((N, C), f32),
    )(q, xw2p, dinv, WlinT, blin2)


def kernel(x, edge_index, edge_weight, W0_1, Wih_1, Whh_1, bih_1, bhh_1,
           W0_2, Wih_2, Whh_2, bih_2, bhh_2, Wlin, blin):
    edge_flat = edge_index.reshape(2 * E)
    deg_parts = _sc_deg(edge_flat, edge_weight)
    W1, W2 = _tc_gru(
        W0_1, Wih_1.T, Whh_1.T, bih_1.reshape(1, 3 * D), bhh_1.reshape(1, 3 * D),
        W0_2, Wih_2.T, Whh_2.T, bih_2.reshape(1, 3 * D), bhh_2.reshape(1, 3 * D),
    )
    xw1p, dinv = _tc_xw1(x, W1, deg_parts.T)
    p = _sc_conv(xw1p, edge_flat, edge_weight)
    xw2p = _tc_mid(p, xw1p, dinv, W2)
    q = _sc_conv(xw2p, edge_flat, edge_weight)
    return _tc_final(q, xw2p, dinv, Wlin.T, blin.reshape(1, C))


# speedup vs baseline: 17.1367x; 1.8955x over previous
import functools

import jax
import jax.numpy as jnp
from jax import lax
from jax.experimental import pallas as pl
from jax.experimental.pallas import tpu as pltpu
from jax.experimental.pallas import tpu_sc as plsc

N = 10000
E = 320000
D = 128
C = 40

NC = 2
NS = 16
NT = NC * NS
EPT = E // NT
K = 80
NCHUNK = EPT // K
SZ = 624
ZR = 104

_mesh = plsc.VectorSubcoreMesh(
    core_axis_name="c", subcore_axis_name="s", num_cores=NC, num_subcores=NS
)
_sc_params = pltpu.CompilerParams(needs_layout_passes=False)

f32 = jnp.float32


@functools.partial(
    pl.kernel,
    out_type=jax.ShapeDtypeStruct((NT, N), f32),
    mesh=_mesh,
    compiler_params=_sc_params,
    scratch_types=[
        pltpu.VMEM((N,), f32),
        pltpu.VMEM((EPT,), jnp.int32),
        pltpu.VMEM((EPT,), f32),
        pltpu.SemaphoreType.DMA,
        pltpu.SemaphoreType.DMA,
    ],
)
def _sc_deg(eflat_ref, ew_ref, out_ref, degl, colb, ewb, semc, semw):
    c = lax.axis_index("c")
    s = lax.axis_index("s")
    wid = c * NS + s

    pltpu.async_copy(eflat_ref.at[pl.ds(E + wid * EPT, EPT)], colb, semc)
    pltpu.async_copy(ew_ref.at[pl.ds(wid * EPT, EPT)], ewb, semw)

    def zero_body(i, _):
        degl[pl.ds(i * 16, 16)] = jnp.zeros((16,), f32)
        return 0

    lax.fori_loop(0, N // 16, zero_body, 0)

    pltpu.make_async_copy(
        eflat_ref.at[pl.ds(E + wid * EPT, EPT)], colb, semc).wait()
    pltpu.make_async_copy(ew_ref.at[pl.ds(wid * EPT, EPT)], ewb, semw).wait()

    def vec_body(i, _):
        cv = colb[pl.ds(i * 16, 16)]
        ev = ewb[pl.ds(i * 16, 16)]
        plsc.addupdate_scatter(degl, [cv], ev)
        return 0

    lax.fori_loop(0, EPT // 16, vec_body, 0)
    pltpu.sync_copy(degl, out_ref.at[wid])


@functools.partial(
    pl.kernel,
    out_type=jax.ShapeDtypeStruct((NC, N, D), f32),
    mesh=_mesh,
    compiler_params=_sc_params,
    scratch_types=[
        pltpu.VMEM_SHARED((N, D), f32),
        pltpu.VMEM((ZR, D), f32),
        pltpu.VMEM((K,), f32),
        pltpu.VMEM((K,), f32),
        pltpu.VMEM((K,), f32),
        pltpu.VMEM((K,), jnp.int32),
        pltpu.VMEM((K,), jnp.int32),
        pltpu.VMEM((K,), jnp.int32),
        pltpu.VMEM((K,), jnp.int32),
        pltpu.VMEM((K,), jnp.int32),
        pltpu.VMEM((K,), jnp.int32),
        pltpu.VMEM((K, D), f32),
        pltpu.VMEM((K, D), f32),
        pltpu.VMEM((K, D), f32),
        pltpu.SemaphoreType.DMA,
        pltpu.SemaphoreType.DMA,
        pltpu.SemaphoreType.DMA,
        pltpu.SemaphoreType.DMA,
        pltpu.SemaphoreType.DMA,
        pltpu.SemaphoreType.DMA,
        pltpu.SemaphoreType.DMA,
        pltpu.SemaphoreType.DMA,
        pltpu.SemaphoreType.DMA,
    ],
)
def _sc_conv(xw_ref, eflat_ref, ew_ref, out_ref, acc, zbuf,
             ewb0, ewb1, ewb2, rowb0, rowb1, rowb2, colb0, colb1, colb2,
             rbuf0, rbuf1, rbuf2, si0, si1, si2, sg0, sg1, sg2,
             ss0, ss1, ss2):
    c = lax.axis_index("c")
    s = lax.axis_index("s")
    wid = c * NS + s
    ewbs = (ewb0, ewb1, ewb2)
    rowbs = (rowb0, rowb1, rowb2)
    colbs = (colb0, colb1, colb2)
    rbufs = (rbuf0, rbuf1, rbuf2)
    sis = (si0, si1, si2)
    sgs = (sg0, sg1, sg2)
    sss = (ss0, ss1, ss2)

    def start_idx(j, b):
        base = wid * EPT + j * K
        pltpu.async_copy(eflat_ref.at[pl.ds(base, K)], rowbs[b], sis[b])
        pltpu.async_copy(eflat_ref.at[pl.ds(E + base, K)], colbs[b], sis[b])
        pltpu.async_copy(ew_ref.at[pl.ds(base, K)], ewbs[b], sis[b])

    def wait_idx(j, b):
        base = wid * EPT + j * K
        pltpu.make_async_copy(
            eflat_ref.at[pl.ds(base, K)], rowbs[b], sis[b]).wait()
        pltpu.make_async_copy(
            eflat_ref.at[pl.ds(E + base, K)], colbs[b], sis[b]).wait()
        pltpu.make_async_copy(
            ew_ref.at[pl.ds(base, K)], ewbs[b], sis[b]).wait()

    def start_gather(b):
        pltpu.async_copy(xw_ref.at[rowbs[b]], rbufs[b], sgs[b])

    def wait_gather(b):
        pltpu.make_async_copy(xw_ref.at[rowbs[b]], rbufs[b], sgs[b]).wait()

    def start_scatter(b):
        pltpu.async_copy(rbufs[b], acc.at[colbs[b]], sss[b], add=True)

    def wait_scatter(b):
        pltpu.make_async_copy(rbufs[b], acc.at[colbs[b]], sss[b]).wait()

    start_idx(0, 0)
    start_idx(1, 1)

    def zrow(r, _):
        for g in range(D // 16):
            zbuf[r, pl.ds(g * 16, 16)] = jnp.zeros((16,), f32)
        return 0

    lax.fori_loop(0, ZR, zrow, 0)

    def zcopy(m, _):
        pltpu.sync_copy(zbuf, acc.at[pl.ds(s * SZ + m * ZR, ZR)])
        return 0

    lax.fori_loop(0, SZ // ZR, zcopy, 0)

    @pl.when(s == NS - 1)
    def _ztail():
        pltpu.sync_copy(zbuf.at[pl.ds(0, 16)], acc.at[pl.ds(NS * SZ, 16)])

    plsc.subcore_barrier()

    wait_idx(0, 0)
    start_gather(0)

    def scale(j, b):
        rb = rbufs[b]
        eb = ewbs[b]

        def edge_body(e, _):
            w = plsc.load_gather(eb, [jnp.full((16,), e, jnp.int32)])
            for g in range(D // 16):
                rb[e, pl.ds(g * 16, 16)] = rb[e, pl.ds(g * 16, 16)] * w
            return 0

        lax.fori_loop(0, K, edge_body, 0)

    def chunk_step(j, b, wait_prev, prefetch, next_gather):
        wait_gather(b)
        scale(j, b)
        start_scatter(b)
        if wait_prev:
            wait_scatter((b + 2) % 3)
        if prefetch:
            start_idx(j + 2, (b + 2) % 3)
        if next_gather:
            wait_idx(j + 1, (b + 1) % 3)
            start_gather((b + 1) % 3)

    chunk_step(0, 0, False, True, True)

    def loop_body(j2, _):
        j = 1 + 3 * j2
        chunk_step(j, 1, True, True, True)
        chunk_step(j + 1, 2, True, True, True)
        chunk_step(j + 2, 0, True, True, True)
        return 0

    lax.fori_loop(0, (NCHUNK - 5) // 3, loop_body, 0)

    chunk_step(NCHUNK - 4, 1, True, True, True)
    chunk_step(NCHUNK - 3, 2, True, True, True)
    chunk_step(NCHUNK - 2, 0, True, False, True)
    chunk_step(NCHUNK - 1, 1, True, False, False)
    wait_scatter(1)

    plsc.subcore_barrier()

    def outcopy(m, _):
        lo = s * SZ + m * ZR
        pltpu.sync_copy(acc.at[pl.ds(lo, ZR)], out_ref.at[c, pl.ds(lo, ZR)])
        return 0

    lax.fori_loop(0, SZ // ZR, outcopy, 0)

    @pl.when(s == NS - 1)
    def _otail():
        pltpu.sync_copy(acc.at[pl.ds(NS * SZ, 16)],
                        out_ref.at[c, pl.ds(NS * SZ, 16)])


def _gru(W0, WihT, WhhT, bih, bhh):
    gi = jnp.dot(W0, WihT, preferred_element_type=f32) + bih
    gh = jnp.dot(W0, WhhT, preferred_element_type=f32) + bhh
    r = jax.nn.sigmoid(gi[:, :D] + gh[:, :D])
    z = jax.nn.sigmoid(gi[:, D:2 * D] + gh[:, D:2 * D])
    n = jnp.tanh(gi[:, 2 * D:] + r * gh[:, 2 * D:])
    return (1.0 - z) * n + z * W0


def _tc_gru_body(W01, WihT1, WhhT1, bih1, bhh1, W02, WihT2, WhhT2, bih2, bhh2,
                 W1_out, W2_out):
    W1_out[...] = _gru(W01[...], WihT1[...], WhhT1[...], bih1[...], bhh1[...])
    W2_out[...] = _gru(W02[...], WihT2[...], WhhT2[...], bih2[...], bhh2[...])


def _tc_gru(W01, WihT1, WhhT1, bih1, bhh1, W02, WihT2, WhhT2, bih2, bhh2):
    return pl.pallas_call(
        _tc_gru_body,
        out_shape=(jax.ShapeDtypeStruct((D, D), f32),
                   jax.ShapeDtypeStruct((D, D), f32)),
    )(W01, WihT1, WhhT1, bih1, bhh1, W02, WihT2, WhhT2, bih2, bhh2)


_RB = 1000
_NG = N // _RB


def _tc_xw1_body(x_ref, W1_ref, degT_ref, xw_ref, dinv_ref):
    deg = jnp.sum(degT_ref[...], axis=1, keepdims=True) + 1.0
    dinv = lax.rsqrt(deg)
    dinv_ref[...] = dinv
    xw_ref[...] = dinv * jnp.dot(x_ref[...], W1_ref[...],
                                 preferred_element_type=f32)


def _tc_xw1(x, W1, deg_partsT):
    return pl.pallas_call(
        _tc_xw1_body,
        grid=(_NG,),
        in_specs=[
            pl.BlockSpec((_RB, D), lambda i: (i, 0)),
            pl.BlockSpec((D, D), lambda i: (0, 0)),
            pl.BlockSpec((_RB, NT), lambda i: (i, 0)),
        ],
        out_specs=(
            pl.BlockSpec((_RB, D), lambda i: (i, 0)),
            pl.BlockSpec((_RB, 1), lambda i: (i, 0)),
        ),
        out_shape=(jax.ShapeDtypeStruct((N, D), f32),
                   jax.ShapeDtypeStruct((N, 1), f32)),
    )(x, W1, deg_partsT)


def _tc_mid_body(p_ref, xw_ref, dinv_ref, W2_ref, out_ref):
    dinv = dinv_ref[...]
    h = jnp.maximum(dinv * (p_ref[0] + p_ref[1] + xw_ref[...]), 0.0)
    out_ref[...] = dinv * jnp.dot(h, W2_ref[...], preferred_element_type=f32)


def _tc_mid(p, xw1p, dinv, W2):
    return pl.pallas_call(
        _tc_mid_body,
        grid=(_NG,),
        in_specs=[
            pl.BlockSpec((NC, _RB, D), lambda i: (0, i, 0)),
            pl.BlockSpec((_RB, D), lambda i: (i, 0)),
            pl.BlockSpec((_RB, 1), lambda i: (i, 0)),
            pl.BlockSpec((D, D), lambda i: (0, 0)),
        ],
        out_specs=pl.BlockSpec((_RB, D), lambda i: (i, 0)),
        out_shape=jax.ShapeDtypeStruct((N, D), f32),
    )(p, xw1p, dinv, W2)


def _tc_final_body(q_ref, xw_ref, dinv_ref, WlinT_ref, blin_ref, out_ref):
    dinv = dinv_ref[...]
    h = jnp.maximum(dinv * (q_ref[0] + q_ref[1] + xw_ref[...]), 0.0)
    logits = jnp.dot(h, WlinT_ref[...], preferred_element_type=f32) + blin_ref[...]
    m = jnp.max(logits, axis=-1, keepdims=True)
    lse = m + jnp.log(jnp.sum(jnp.exp(logits - m), axis=-1, keepdims=True))
    out_ref[...] = logits - lse


def _tc_final(q, xw2p, dinv, WlinT, blin2):
    return pl.pallas_call(
        _tc_final_body,
        grid=(_NG,),
        in_specs=[
            pl.BlockSpec((NC, _RB, D), lambda i: (0, i, 0)),
            pl.BlockSpec((_RB, D), lambda i: (i, 0)),
            pl.BlockSpec((_RB, 1), lambda i: (i, 0)),
            pl.BlockSpec((D, C), lambda i: (0, 0)),
            pl.BlockSpec((1, C), lambda i: (0, 0)),
        ],
        out_specs=pl.BlockSpec((_RB, C), lambda i: (i, 0)),
        out_shape=jax.ShapeDtypeStruct((N, C), f32),
    )(q, xw2p, dinv, WlinT, blin2)


def kernel(x, edge_index, edge_weight, W0_1, Wih_1, Whh_1, bih_1, bhh_1,
           W0_2, Wih_2, Whh_2, bih_2, bhh_2, Wlin, blin):
    edge_flat = edge_index.reshape(2 * E)

    deg_parts = _sc_deg(edge_flat, edge_weight)
    W1, W2 = _tc_gru(
        W0_1, Wih_1.T, Whh_1.T, bih_1.reshape(1, 3 * D), bhh_1.reshape(1, 3 * D),
        W0_2, Wih_2.T, Whh_2.T, bih_2.reshape(1, 3 * D), bhh_2.reshape(1, 3 * D),
    )
    xw1p, dinv = _tc_xw1(x, W1, deg_parts.T)
    p = _sc_conv(xw1p, edge_flat, edge_weight)
    xw2p = _tc_mid(p, xw1p, dinv, W2)
    q = _sc_conv(xw2p, edge_flat, edge_weight)
    return _tc_final(q, xw2p, dinv, Wlin.T, blin.reshape(1, C))


# speedup vs baseline: 19.9395x; 1.1636x over previous
import functools

import jax
import jax.numpy as jnp
from jax import lax
from jax.experimental import pallas as pl
from jax.experimental.pallas import tpu as pltpu
from jax.experimental.pallas import tpu_sc as plsc

N = 10000
E = 320000
D = 128
C = 40

NC = 2
NS = 16
NT = NC * NS
EPT = E // NT
K = 80
NCHUNK = EPT // K
SZ = 624
ZR = 104

_mesh = plsc.VectorSubcoreMesh(
    core_axis_name="c", subcore_axis_name="s", num_cores=NC, num_subcores=NS
)
_sc_params = pltpu.CompilerParams(needs_layout_passes=False)

f32 = jnp.float32


@functools.partial(
    pl.kernel,
    out_type=jax.ShapeDtypeStruct((NT, N), f32),
    mesh=_mesh,
    compiler_params=_sc_params,
    scratch_types=[
        pltpu.VMEM((N,), f32),
        pltpu.VMEM((EPT,), jnp.int32),
        pltpu.VMEM((EPT,), f32),
        pltpu.SemaphoreType.DMA,
        pltpu.SemaphoreType.DMA,
    ],
)
def _sc_deg(eflat_ref, ew_ref, out_ref, degl, colb, ewb, semc, semw):
    c = lax.axis_index("c")
    s = lax.axis_index("s")
    wid = c * NS + s

    pltpu.async_copy(eflat_ref.at[pl.ds(E + wid * EPT, EPT)], colb, semc)
    pltpu.async_copy(ew_ref.at[pl.ds(wid * EPT, EPT)], ewb, semw)

    def zero_body(i, _):
        degl[pl.ds(i * 16, 16)] = jnp.zeros((16,), f32)
        return 0

    lax.fori_loop(0, N // 16, zero_body, 0)

    pltpu.make_async_copy(
        eflat_ref.at[pl.ds(E + wid * EPT, EPT)], colb, semc).wait()
    pltpu.make_async_copy(ew_ref.at[pl.ds(wid * EPT, EPT)], ewb, semw).wait()

    def vec_body(i, _):
        cv = colb[pl.ds(i * 16, 16)]
        ev = ewb[pl.ds(i * 16, 16)]
        plsc.addupdate_scatter(degl, [cv], ev)
        return 0

    lax.fori_loop(0, EPT // 16, vec_body, 0)
    pltpu.sync_copy(degl, out_ref.at[wid])


@functools.partial(
    pl.kernel,
    out_type=jax.ShapeDtypeStruct((NC, N, D), f32),
    mesh=_mesh,
    compiler_params=_sc_params,
    scratch_types=[
        pltpu.VMEM_SHARED((N, D), f32),
        pltpu.VMEM((ZR, D), f32),
        pltpu.VMEM((K,), f32),
        pltpu.VMEM((K,), f32),
        pltpu.VMEM((K,), f32),
        pltpu.VMEM((K,), jnp.int32),
        pltpu.VMEM((K,), jnp.int32),
        pltpu.VMEM((K,), jnp.int32),
        pltpu.VMEM((K,), jnp.int32),
        pltpu.VMEM((K,), jnp.int32),
        pltpu.VMEM((K,), jnp.int32),
        pltpu.VMEM((K, D), f32),
        pltpu.VMEM((K, D), f32),
        pltpu.VMEM((K, D), f32),
        pltpu.SemaphoreType.DMA,
        pltpu.SemaphoreType.DMA,
        pltpu.SemaphoreType.DMA,
        pltpu.SemaphoreType.DMA,
        pltpu.SemaphoreType.DMA,
        pltpu.SemaphoreType.DMA,
        pltpu.SemaphoreType.DMA,
        pltpu.SemaphoreType.DMA,
        pltpu.SemaphoreType.DMA,
    ],
)
def _sc_conv(xw_ref, eflat_ref, ew_ref, out_ref, acc, zbuf,
             ewb0, ewb1, ewb2, rowb0, rowb1, rowb2, colb0, colb1, colb2,
             rbuf0, rbuf1, rbuf2, si0, si1, si2, sg0, sg1, sg2,
             ss0, ss1, ss2):
    c = lax.axis_index("c")
    s = lax.axis_index("s")
    wid = c * NS + s
    ewbs = (ewb0, ewb1, ewb2)
    rowbs = (rowb0, rowb1, rowb2)
    colbs = (colb0, colb1, colb2)
    rbufs = (rbuf0, rbuf1, rbuf2)
    sis = (si0, si1, si2)
    sgs = (sg0, sg1, sg2)
    sss = (ss0, ss1, ss2)

    def start_idx(j, b):
        base = wid * EPT + j * K
        pltpu.async_copy(eflat_ref.at[pl.ds(base, K)], rowbs[b], sis[b])
        pltpu.async_copy(eflat_ref.at[pl.ds(E + base, K)], colbs[b], sis[b])
        pltpu.async_copy(ew_ref.at[pl.ds(base, K)], ewbs[b], sis[b])

    def wait_idx(j, b):
        base = wid * EPT + j * K
        pltpu.make_async_copy(
            eflat_ref.at[pl.ds(base, K)], rowbs[b], sis[b]).wait()
        pltpu.make_async_copy(
            eflat_ref.at[pl.ds(E + base, K)], colbs[b], sis[b]).wait()
        pltpu.make_async_copy(
            ew_ref.at[pl.ds(base, K)], ewbs[b], sis[b]).wait()

    def start_gather(b):
        pltpu.async_copy(xw_ref.at[rowbs[b]], rbufs[b], sgs[b])

    def wait_gather(b):
        pltpu.make_async_copy(xw_ref.at[rowbs[b]], rbufs[b], sgs[b]).wait()

    def start_scatter(b):
        pltpu.async_copy(rbufs[b], acc.at[colbs[b]], sss[b], add=True)

    def wait_scatter(b):
        pltpu.make_async_copy(rbufs[b], acc.at[colbs[b]], sss[b]).wait()

    start_idx(0, 0)
    start_idx(1, 1)

    def zrow(r, _):
        for g in range(D // 16):
            zbuf[r, pl.ds(g * 16, 16)] = jnp.zeros((16,), f32)
        return 0

    lax.fori_loop(0, ZR, zrow, 0)

    def zcopy(m, _):
        pltpu.sync_copy(zbuf, acc.at[pl.ds(s * SZ + m * ZR, ZR)])
        return 0

    lax.fori_loop(0, SZ // ZR, zcopy, 0)

    @pl.when(s == NS - 1)
    def _ztail():
        pltpu.sync_copy(zbuf.at[pl.ds(0, 16)], acc.at[pl.ds(NS * SZ, 16)])

    plsc.subcore_barrier()

    wait_idx(0, 0)
    start_gather(0)

    def scale(j, b):
        rb = rbufs[b]
        eb = ewbs[b]

        def edge_body(i, _):
            e0 = i * 4
            ws = [plsc.load_gather(eb, [jnp.full((16,), e0 + u, jnp.int32)])
                  for u in range(4)]
            for g in range(D // 16):
                for u in range(4):
                    rb[e0 + u, pl.ds(g * 16, 16)] = (
                        rb[e0 + u, pl.ds(g * 16, 16)] * ws[u])
            return 0

        lax.fori_loop(0, K // 4, edge_body, 0)

    def chunk_step(j, b, wait_prev, prefetch, next_gather):
        wait_gather(b)
        scale(j, b)
        start_scatter(b)
        if wait_prev:
            wait_scatter((b + 2) % 3)
        if prefetch:
            start_idx(j + 2, (b + 2) % 3)
        if next_gather:
            wait_idx(j + 1, (b + 1) % 3)
            start_gather((b + 1) % 3)

    chunk_step(0, 0, False, True, True)

    def loop_body(j2, _):
        j = 1 + 3 * j2
        chunk_step(j, 1, True, True, True)
        chunk_step(j + 1, 2, True, True, True)
        chunk_step(j + 2, 0, True, True, True)
        return 0

    lax.fori_loop(0, (NCHUNK - 5) // 3, loop_body, 0)

    chunk_step(NCHUNK - 4, 1, True, True, True)
    chunk_step(NCHUNK - 3, 2, True, True, True)
    chunk_step(NCHUNK - 2, 0, True, False, True)
    chunk_step(NCHUNK - 1, 1, True, False, False)
    wait_scatter(1)

    plsc.subcore_barrier()

    def outcopy(m, _):
        lo = s * SZ + m * ZR
        pltpu.sync_copy(acc.at[pl.ds(lo, ZR)], out_ref.at[c, pl.ds(lo, ZR)])
        return 0

    lax.fori_loop(0, SZ // ZR, outcopy, 0)

    @pl.when(s == NS - 1)
    def _otail():
        pltpu.sync_copy(acc.at[pl.ds(NS * SZ, 16)],
                        out_ref.at[c, pl.ds(NS * SZ, 16)])


def _gru(W0, WihT, WhhT, bih, bhh):
    gi = jnp.dot(W0, WihT, preferred_element_type=f32) + bih
    gh = jnp.dot(W0, WhhT, preferred_element_type=f32) + bhh
    r = jax.nn.sigmoid(gi[:, :D] + gh[:, :D])
    z = jax.nn.sigmoid(gi[:, D:2 * D] + gh[:, D:2 * D])
    n = jnp.tanh(gi[:, 2 * D:] + r * gh[:, 2 * D:])
    return (1.0 - z) * n + z * W0


def _tc_gru_body(W01, WihT1, WhhT1, bih1, bhh1, W02, WihT2, WhhT2, bih2, bhh2,
                 W1_out, W2_out):
    W1_out[...] = _gru(W01[...], WihT1[...], WhhT1[...], bih1[...], bhh1[...])
    W2_out[...] = _gru(W02[...], WihT2[...], WhhT2[...], bih2[...], bhh2[...])


def _tc_gru(W01, WihT1, WhhT1, bih1, bhh1, W02, WihT2, WhhT2, bih2, bhh2):
    return pl.pallas_call(
        _tc_gru_body,
        out_shape=(jax.ShapeDtypeStruct((D, D), f32),
                   jax.ShapeDtypeStruct((D, D), f32)),
    )(W01, WihT1, WhhT1, bih1, bhh1, W02, WihT2, WhhT2, bih2, bhh2)


_RB = 1000
_NG = N // _RB


def _tc_xw1_body(x_ref, W1_ref, degT_ref, xw_ref, dinv_ref):
    deg = jnp.sum(degT_ref[...], axis=1, keepdims=True) + 1.0
    dinv = lax.rsqrt(deg)
    dinv_ref[...] = dinv
    xw_ref[...] = dinv * jnp.dot(x_ref[...], W1_ref[...],
                                 preferred_element_type=f32)


def _tc_xw1(x, W1, deg_partsT):
    return pl.pallas_call(
        _tc_xw1_body,
        grid=(_NG,),
        in_specs=[
            pl.BlockSpec((_RB, D), lambda i: (i, 0)),
            pl.BlockSpec((D, D), lambda i: (0, 0)),
            pl.BlockSpec((_RB, NT), lambda i: (i, 0)),
        ],
        out_specs=(
            pl.BlockSpec((_RB, D), lambda i: (i, 0)),
            pl.BlockSpec((_RB, 1), lambda i: (i, 0)),
        ),
        out_shape=(jax.ShapeDtypeStruct((N, D), f32),
                   jax.ShapeDtypeStruct((N, 1), f32)),
    )(x, W1, deg_partsT)


def _tc_mid_body(p_ref, xw_ref, dinv_ref, W2_ref, out_ref):
    dinv = dinv_ref[...]
    h = jnp.maximum(dinv * (p_ref[0] + p_ref[1] + xw_ref[...]), 0.0)
    out_ref[...] = dinv * jnp.dot(h, W2_ref[...], preferred_element_type=f32)


def _tc_mid(p, xw1p, dinv, W2):
    return pl.pallas_call(
        _tc_mid_body,
        grid=(_NG,),
        in_specs=[
            pl.BlockSpec((NC, _RB, D), lambda i: (0, i, 0)),
            pl.BlockSpec((_RB, D), lambda i: (i, 0)),
            pl.BlockSpec((_RB, 1), lambda i: (i, 0)),
            pl.BlockSpec((D, D), lambda i: (0, 0)),
        ],
        out_specs=pl.BlockSpec((_RB, D), lambda i: (i, 0)),
        out_shape=jax.ShapeDtypeStruct((N, D), f32),
    )(p, xw1p, dinv, W2)


def _tc_final_body(q_ref, xw_ref, dinv_ref, WlinT_ref, blin_ref, out_ref):
    dinv = dinv_ref[...]
    h = jnp.maximum(dinv * (q_ref[0] + q_ref[1] + xw_ref[...]), 0.0)
    logits = jnp.dot(h, WlinT_ref[...], preferred_element_type=f32) + blin_ref[...]
    m = jnp.max(logits, axis=-1, keepdims=True)
    lse = m + jnp.log(jnp.sum(jnp.exp(logits - m), axis=-1, keepdims=True))
    out_ref[...] = logits - lse


def _tc_final(q, xw2p, dinv, WlinT, blin2):
    return pl.pallas_call(
        _tc_final_body,
        grid=(_NG,),
        in_specs=[
            pl.BlockSpec((NC, _RB, D), lambda i: (0, i, 0)),
            pl.BlockSpec((_RB, D), lambda i: (i, 0)),
            pl.BlockSpec((_RB, 1), lambda i: (i, 0)),
            pl.BlockSpec((D, C), lambda i: (0, 0)),
            pl.BlockSpec((1, C), lambda i: (0, 0)),
        ],
        out_specs=pl.BlockSpec((_RB, C), lambda i: (i, 0)),
        out_shape=jax.ShapeDtypeStruct((N, C), f32),
    )(q, xw2p, dinv, WlinT, blin2)


def kernel(x, edge_index, edge_weight, W0_1, Wih_1, Whh_1, bih_1, bhh_1,
           W0_2, Wih_2, Whh_2, bih_2, bhh_2, Wlin, blin):
    edge_flat = edge_index.reshape(2 * E)

    deg_parts = _sc_deg(edge_flat, edge_weight)
    W1, W2 = _tc_gru(
        W0_1, Wih_1.T, Whh_1.T, bih_1.reshape(1, 3 * D), bhh_1.reshape(1, 3 * D),
        W0_2, Wih_2.T, Whh_2.T, bih_2.reshape(1, 3 * D), bhh_2.reshape(1, 3 * D),
    )
    xw1p, dinv = _tc_xw1(x, W1, deg_parts.T)
    p = _sc_conv(xw1p, edge_flat, edge_weight)
    xw2p = _tc_mid(p, xw1p, dinv, W2)
    q = _sc_conv(xw2p, edge_flat, edge_weight)
    return _tc_final(q, xw2p, dinv, Wlin.T, blin.reshape(1, C))


# speedup vs baseline: 20.5511x; 1.0307x over previous
import functools

import jax
import jax.numpy as jnp
from jax import lax
from jax.experimental import pallas as pl
from jax.experimental.pallas import tpu as pltpu
from jax.experimental.pallas import tpu_sc as plsc

N = 10000
E = 320000
D = 128
C = 40

NC = 2
NS = 16
NT = NC * NS
EPT = E // NT
K = 80
NCHUNK = EPT // K
SZ = 624
ZR = 104

_mesh = plsc.VectorSubcoreMesh(
    core_axis_name="c", subcore_axis_name="s", num_cores=NC, num_subcores=NS
)
_sc_params = pltpu.CompilerParams(needs_layout_passes=False)

f32 = jnp.float32


@functools.partial(
    pl.kernel,
    out_type=jax.ShapeDtypeStruct((NT, N), f32),
    mesh=_mesh,
    compiler_params=_sc_params,
    scratch_types=[
        pltpu.VMEM((N,), f32),
        pltpu.VMEM((EPT,), jnp.int32),
        pltpu.VMEM((EPT,), f32),
        pltpu.SemaphoreType.DMA,
        pltpu.SemaphoreType.DMA,
    ],
)
def _sc_deg(eflat_ref, ew_ref, out_ref, degl, colb, ewb, semc, semw):
    c = lax.axis_index("c")
    s = lax.axis_index("s")
    wid = c * NS + s

    pltpu.async_copy(eflat_ref.at[pl.ds(E + wid * EPT, EPT)], colb, semc)
    pltpu.async_copy(ew_ref.at[pl.ds(wid * EPT, EPT)], ewb, semw)

    def zero_body(i, _):
        degl[pl.ds(i * 16, 16)] = jnp.zeros((16,), f32)
        return 0

    lax.fori_loop(0, N // 16, zero_body, 0)

    pltpu.make_async_copy(
        eflat_ref.at[pl.ds(E + wid * EPT, EPT)], colb, semc).wait()
    pltpu.make_async_copy(ew_ref.at[pl.ds(wid * EPT, EPT)], ewb, semw).wait()

    def vec_body(i, _):
        cv = colb[pl.ds(i * 16, 16)]
        ev = ewb[pl.ds(i * 16, 16)]
        plsc.addupdate_scatter(degl, [cv], ev)
        return 0

    lax.fori_loop(0, EPT // 16, vec_body, 0)
    pltpu.sync_copy(degl, out_ref.at[wid])


@functools.partial(
    pl.kernel,
    out_type=jax.ShapeDtypeStruct((NC, N, D), f32),
    mesh=_mesh,
    compiler_params=_sc_params,
    scratch_types=[
        pltpu.VMEM_SHARED((N, D), f32),
        pltpu.VMEM((ZR, D), f32),
        pltpu.VMEM((K,), f32),
        pltpu.VMEM((K,), f32),
        pltpu.VMEM((K,), f32),
        pltpu.VMEM((K,), jnp.int32),
        pltpu.VMEM((K,), jnp.int32),
        pltpu.VMEM((K,), jnp.int32),
        pltpu.VMEM((K,), jnp.int32),
        pltpu.VMEM((K,), jnp.int32),
        pltpu.VMEM((K,), jnp.int32),
        pltpu.VMEM((K, D), f32),
        pltpu.VMEM((K, D), f32),
        pltpu.VMEM((K, D), f32),
        pltpu.SemaphoreType.DMA,
        pltpu.SemaphoreType.DMA,
        pltpu.SemaphoreType.DMA,
        pltpu.SemaphoreType.DMA,
        pltpu.SemaphoreType.DMA,
        pltpu.SemaphoreType.DMA,
        pltpu.SemaphoreType.DMA,
        pltpu.SemaphoreType.DMA,
        pltpu.SemaphoreType.DMA,
    ],
)
def _sc_conv(xw_ref, eflat_ref, ew_ref, out_ref, acc, zbuf,
             ewb0, ewb1, ewb2, rowb0, rowb1, rowb2, colb0, colb1, colb2,
             rbuf0, rbuf1, rbuf2, si0, si1, si2, sg0, sg1, sg2,
             ss0, ss1, ss2):
    c = lax.axis_index("c")
    s = lax.axis_index("s")
    wid = c * NS + s
    ewbs = (ewb0, ewb1, ewb2)
    rowbs = (rowb0, rowb1, rowb2)
    colbs = (colb0, colb1, colb2)
    rbufs = (rbuf0, rbuf1, rbuf2)
    sis = (si0, si1, si2)
    sgs = (sg0, sg1, sg2)
    sss = (ss0, ss1, ss2)

    def start_idx(j, b):
        base = wid * EPT + j * K
        pltpu.async_copy(eflat_ref.at[pl.ds(base, K)], rowbs[b], sis[b])
        pltpu.async_copy(eflat_ref.at[pl.ds(E + base, K)], colbs[b], sis[b])
        pltpu.async_copy(ew_ref.at[pl.ds(base, K)], ewbs[b], sis[b])

    def wait_idx(j, b):
        base = wid * EPT + j * K
        pltpu.make_async_copy(
            eflat_ref.at[pl.ds(base, K)], rowbs[b], sis[b]).wait()
        pltpu.make_async_copy(
            eflat_ref.at[pl.ds(E + base, K)], colbs[b], sis[b]).wait()
        pltpu.make_async_copy(
            ew_ref.at[pl.ds(base, K)], ewbs[b], sis[b]).wait()

    def start_gather(b):
        pltpu.async_copy(xw_ref.at[rowbs[b]], rbufs[b], sgs[b])

    def wait_gather(b):
        pltpu.make_async_copy(xw_ref.at[rowbs[b]], rbufs[b], sgs[b]).wait()

    def start_scatter(b):
        pltpu.async_copy(rbufs[b], acc.at[colbs[b]], sss[b], add=True)

    def wait_scatter(b):
        pltpu.make_async_copy(rbufs[b], acc.at[colbs[b]], sss[b]).wait()

    start_idx(0, 0)
    start_idx(1, 1)

    def zrow(r, _):
        for g in range(D // 16):
            zbuf[r, pl.ds(g * 16, 16)] = jnp.zeros((16,), f32)
        return 0

    lax.fori_loop(0, ZR, zrow, 0)

    def zcopy(m, _):
        pltpu.sync_copy(zbuf, acc.at[pl.ds(s * SZ + m * ZR, ZR)])
        return 0

    lax.fori_loop(0, SZ // ZR, zcopy, 0)

    @pl.when(s == NS - 1)
    def _ztail():
        pltpu.sync_copy(zbuf.at[pl.ds(0, 16)], acc.at[pl.ds(NS * SZ, 16)])

    plsc.subcore_barrier()

    wait_idx(0, 0)
    start_gather(0)

    def scale(j, b):
        rb = rbufs[b]
        eb = ewbs[b]

        @plsc.parallel_loop(0, K, 1, unroll=4)
        def _edge(e):
            w = plsc.load_gather(eb, [jnp.full((16,), e, jnp.int32)])
            for g in range(D // 16):
                rb[e, pl.ds(g * 16, 16)] = rb[e, pl.ds(g * 16, 16)] * w

    def chunk_step(j, b, wait_prev, prefetch, next_gather):
        wait_gather(b)
        scale(j, b)
        start_scatter(b)
        if wait_prev:
            wait_scatter((b + 2) % 3)
        if prefetch:
            start_idx(j + 2, (b + 2) % 3)
        if next_gather:
            wait_idx(j + 1, (b + 1) % 3)
            start_gather((b + 1) % 3)

    chunk_step(0, 0, False, True, True)

    def loop_body(j2, _):
        j = 1 + 3 * j2
        chunk_step(j, 1, True, True, True)
        chunk_step(j + 1, 2, True, True, True)
        chunk_step(j + 2, 0, True, True, True)
        return 0

    lax.fori_loop(0, (NCHUNK - 5) // 3, loop_body, 0)

    chunk_step(NCHUNK - 4, 1, True, True, True)
    chunk_step(NCHUNK - 3, 2, True, True, True)
    chunk_step(NCHUNK - 2, 0, True, False, True)
    chunk_step(NCHUNK - 1, 1, True, False, False)
    wait_scatter(1)

    plsc.subcore_barrier()

    def outcopy(m, _):
        lo = s * SZ + m * ZR
        pltpu.sync_copy(acc.at[pl.ds(lo, ZR)], out_ref.at[c, pl.ds(lo, ZR)])
        return 0

    lax.fori_loop(0, SZ // ZR, outcopy, 0)

    @pl.when(s == NS - 1)
    def _otail():
        pltpu.sync_copy(acc.at[pl.ds(NS * SZ, 16)],
                        out_ref.at[c, pl.ds(NS * SZ, 16)])


def _gru(W0, WihT, WhhT, bih, bhh):
    gi = jnp.dot(W0, WihT, preferred_element_type=f32) + bih
    gh = jnp.dot(W0, WhhT, preferred_element_type=f32) + bhh
    r = jax.nn.sigmoid(gi[:, :D] + gh[:, :D])
    z = jax.nn.sigmoid(gi[:, D:2 * D] + gh[:, D:2 * D])
    n = jnp.tanh(gi[:, 2 * D:] + r * gh[:, 2 * D:])
    return (1.0 - z) * n + z * W0


def _tc_gru_body(W01, WihT1, WhhT1, bih1, bhh1, W02, WihT2, WhhT2, bih2, bhh2,
                 W1_out, W2_out):
    W1_out[...] = _gru(W01[...], WihT1[...], WhhT1[...], bih1[...], bhh1[...])
    W2_out[...] = _gru(W02[...], WihT2[...], WhhT2[...], bih2[...], bhh2[...])


def _tc_gru(W01, WihT1, WhhT1, bih1, bhh1, W02, WihT2, WhhT2, bih2, bhh2):
    return pl.pallas_call(
        _tc_gru_body,
        out_shape=(jax.ShapeDtypeStruct((D, D), f32),
                   jax.ShapeDtypeStruct((D, D), f32)),
    )(W01, WihT1, WhhT1, bih1, bhh1, W02, WihT2, WhhT2, bih2, bhh2)


_RB = 1000
_NG = N // _RB


def _tc_xw1_body(x_ref, W1_ref, degT_ref, xw_ref, dinv_ref):
    deg = jnp.sum(degT_ref[...], axis=1, keepdims=True) + 1.0
    dinv = lax.rsqrt(deg)
    dinv_ref[...] = dinv
    xw_ref[...] = dinv * jnp.dot(x_ref[...], W1_ref[...],
                                 preferred_element_type=f32)


def _tc_xw1(x, W1, deg_partsT):
    return pl.pallas_call(
        _tc_xw1_body,
        grid=(_NG,),
        in_specs=[
            pl.BlockSpec((_RB, D), lambda i: (i, 0)),
            pl.BlockSpec((D, D), lambda i: (0, 0)),
            pl.BlockSpec((_RB, NT), lambda i: (i, 0)),
        ],
        out_specs=(
            pl.BlockSpec((_RB, D), lambda i: (i, 0)),
            pl.BlockSpec((_RB, 1), lambda i: (i, 0)),
        ),
        out_shape=(jax.ShapeDtypeStruct((N, D), f32),
                   jax.ShapeDtypeStruct((N, 1), f32)),
    )(x, W1, deg_partsT)


def _tc_mid_body(p_ref, xw_ref, dinv_ref, W2_ref, out_ref):
    dinv = dinv_ref[...]
    h = jnp.maximum(dinv * (p_ref[0] + p_ref[1] + xw_ref[...]), 0.0)
    out_ref[...] = dinv * jnp.dot(h, W2_ref[...], preferred_element_type=f32)


def _tc_mid(p, xw1p, dinv, W2):
    return pl.pallas_call(
        _tc_mid_body,
        grid=(_NG,),
        in_specs=[
            pl.BlockSpec((NC, _RB, D), lambda i: (0, i, 0)),
            pl.BlockSpec((_RB, D), lambda i: (i, 0)),
            pl.BlockSpec((_RB, 1), lambda i: (i, 0)),
            pl.BlockSpec((D, D), lambda i: (0, 0)),
        ],
        out_specs=pl.BlockSpec((_RB, D), lambda i: (i, 0)),
        out_shape=jax.ShapeDtypeStruct((N, D), f32),
    )(p, xw1p, dinv, W2)


def _tc_final_body(q_ref, xw_ref, dinv_ref, WlinT_ref, blin_ref, out_ref):
    dinv = dinv_ref[...]
    h = jnp.maximum(dinv * (q_ref[0] + q_ref[1] + xw_ref[...]), 0.0)
    logits = jnp.dot(h, WlinT_ref[...], preferred_element_type=f32) + blin_ref[...]
    m = jnp.max(logits, axis=-1, keepdims=True)
    lse = m + jnp.log(jnp.sum(jnp.exp(logits - m), axis=-1, keepdims=True))
    out_ref[...] = logits - lse


def _tc_final(q, xw2p, dinv, WlinT, blin2):
    return pl.pallas_call(
        _tc_final_body,
        grid=(_NG,),
        in_specs=[
            pl.BlockSpec((NC, _RB, D), lambda i: (0, i, 0)),
            pl.BlockSpec((_RB, D), lambda i: (i, 0)),
            pl.BlockSpec((_RB, 1), lambda i: (i, 0)),
            pl.BlockSpec((D, C), lambda i: (0, 0)),
            pl.BlockSpec((1, C), lambda i: (0, 0)),
        ],
        out_specs=pl.BlockSpec((_RB, C), lambda i: (i, 0)),
        out_shape=jax.ShapeDtypeStruct((N, C), f32),
    )(q, xw2p, dinv, WlinT, blin2)


def kernel(x, edge_index, edge_weight, W0_1, Wih_1, Whh_1, bih_1, bhh_1,
           W0_2, Wih_2, Whh_2, bih_2, bhh_2, Wlin, blin):
    edge_flat = edge_index.reshape(2 * E)

    deg_parts = _sc_deg(edge_flat, edge_weight)
    W1, W2 = _tc_gru(
        W0_1, Wih_1.T, Whh_1.T, bih_1.reshape(1, 3 * D), bhh_1.reshape(1, 3 * D),
        W0_2, Wih_2.T, Whh_2.T, bih_2.reshape(1, 3 * D), bhh_2.reshape(1, 3 * D),
    )
    xw1p, dinv = _tc_xw1(x, W1, deg_parts.T)
    p = _sc_conv(xw1p, edge_flat, edge_weight)
    xw2p = _tc_mid(p, xw1p, dinv, W2)
    q = _sc_conv(xw2p, edge_flat, edge_weight)
    return _tc_final(q, xw2p, dinv, Wlin.T, blin.reshape(1, C))


# speedup vs baseline: 25.8203x; 1.2564x over previous
import functools

import jax
import jax.numpy as jnp
from jax import lax
from jax.experimental import pallas as pl
from jax.experimental.pallas import tpu as pltpu
from jax.experimental.pallas import tpu_sc as plsc

N = 10000
E = 320000
D = 128
C = 40

NC = 2
NS = 16
NT = NC * NS
EPT = E // NT
K = 80
NCHUNK = EPT // K
SZ = 624
ZR = 104

_mesh = plsc.VectorSubcoreMesh(
    core_axis_name="c", subcore_axis_name="s", num_cores=NC, num_subcores=NS
)
_sc_params = pltpu.CompilerParams(needs_layout_passes=False)

f32 = jnp.float32


@functools.partial(
    pl.kernel,
    out_type=jax.ShapeDtypeStruct((NT, N), f32),
    mesh=_mesh,
    compiler_params=_sc_params,
    scratch_types=[
        pltpu.VMEM((N,), f32),
        pltpu.VMEM((EPT,), jnp.int32),
        pltpu.VMEM((EPT,), f32),
        pltpu.SemaphoreType.DMA,
        pltpu.SemaphoreType.DMA,
    ],
)
def _sc_deg(eflat_ref, ew_ref, out_ref, degl, colb, ewb, semc, semw):
    c = lax.axis_index("c")
    s = lax.axis_index("s")
    wid = c * NS + s

    pltpu.async_copy(eflat_ref.at[pl.ds(E + wid * EPT, EPT)], colb, semc)
    pltpu.async_copy(ew_ref.at[pl.ds(wid * EPT, EPT)], ewb, semw)

    def zero_body(i, _):
        degl[pl.ds(i * 16, 16)] = jnp.zeros((16,), f32)
        return 0

    lax.fori_loop(0, N // 16, zero_body, 0)

    pltpu.make_async_copy(
        eflat_ref.at[pl.ds(E + wid * EPT, EPT)], colb, semc).wait()
    pltpu.make_async_copy(ew_ref.at[pl.ds(wid * EPT, EPT)], ewb, semw).wait()

    def vec_body(i, _):
        cv = colb[pl.ds(i * 16, 16)]
        ev = ewb[pl.ds(i * 16, 16)]
        plsc.addupdate_scatter(degl, [cv], ev)
        return 0

    lax.fori_loop(0, EPT // 16, vec_body, 0)
    pltpu.sync_copy(degl, out_ref.at[wid])


@functools.partial(
    pl.kernel,
    out_type=jax.ShapeDtypeStruct((NC, N, D), f32),
    mesh=_mesh,
    compiler_params=_sc_params,
    scratch_types=[
        pltpu.VMEM_SHARED((N, D), f32),
        pltpu.VMEM((ZR, D), f32),
        pltpu.VMEM((K,), f32),
        pltpu.VMEM((K,), f32),
        pltpu.VMEM((K,), f32),
        pltpu.VMEM((K,), jnp.int32),
        pltpu.VMEM((K,), jnp.int32),
        pltpu.VMEM((K,), jnp.int32),
        pltpu.VMEM((K,), jnp.int32),
        pltpu.VMEM((K,), jnp.int32),
        pltpu.VMEM((K,), jnp.int32),
        pltpu.VMEM((K, D), f32),
        pltpu.VMEM((K, D), f32),
        pltpu.VMEM((K, D), f32),
        pltpu.SemaphoreType.DMA,
        pltpu.SemaphoreType.DMA,
        pltpu.SemaphoreType.DMA,
        pltpu.SemaphoreType.DMA,
        pltpu.SemaphoreType.DMA,
        pltpu.SemaphoreType.DMA,
        pltpu.SemaphoreType.DMA,
        pltpu.SemaphoreType.DMA,
        pltpu.SemaphoreType.DMA,
    ],
)
def _sc_conv(xw_ref, eflat_ref, ew_ref, out_ref, acc, zbuf,
             ewb0, ewb1, ewb2, rowb0, rowb1, rowb2, colb0, colb1, colb2,
             rbuf0, rbuf1, rbuf2, si0, si1, si2, sg0, sg1, sg2,
             ss0, ss1, ss2):
    c = lax.axis_index("c")
    s = lax.axis_index("s")
    wid = c * NS + s
    ewbs = (ewb0, ewb1, ewb2)
    rowbs = (rowb0, rowb1, rowb2)
    colbs = (colb0, colb1, colb2)
    rbufs = (rbuf0, rbuf1, rbuf2)
    sis = (si0, si1, si2)
    sgs = (sg0, sg1, sg2)
    sss = (ss0, ss1, ss2)

    def start_idx(j, b):
        base = wid * EPT + j * K
        pltpu.async_copy(eflat_ref.at[pl.ds(base, K)], rowbs[b], sis[b])
        pltpu.async_copy(eflat_ref.at[pl.ds(E + base, K)], colbs[b], sis[b])
        pltpu.async_copy(ew_ref.at[pl.ds(base, K)], ewbs[b], sis[b])

    def wait_idx(j, b):
        base = wid * EPT + j * K
        pltpu.make_async_copy(
            eflat_ref.at[pl.ds(base, K)], rowbs[b], sis[b]).wait()
        pltpu.make_async_copy(
            eflat_ref.at[pl.ds(E + base, K)], colbs[b], sis[b]).wait()
        pltpu.make_async_copy(
            ew_ref.at[pl.ds(base, K)], ewbs[b], sis[b]).wait()

    def start_gather(b):
        pltpu.async_copy(xw_ref.at[rowbs[b]], rbufs[b], sgs[b])

    def wait_gather(b):
        pltpu.make_async_copy(xw_ref.at[rowbs[b]], rbufs[b], sgs[b]).wait()

    def start_scatter(b):
        pltpu.async_copy(rbufs[b], acc.at[colbs[b]], sss[b], add=True)

    def wait_scatter(b):
        pltpu.make_async_copy(rbufs[b], acc.at[colbs[b]], sss[b]).wait()

    start_idx(0, 0)
    start_idx(1, 1)

    def zrow(r, _):
        for g in range(D // 16):
            zbuf[r, pl.ds(g * 16, 16)] = jnp.zeros((16,), f32)
        return 0

    lax.fori_loop(0, ZR, zrow, 0)

    def zcopy(m, _):
        pltpu.sync_copy(zbuf, acc.at[pl.ds(s * SZ + m * ZR, ZR)])
        return 0

    lax.fori_loop(0, SZ // ZR, zcopy, 0)

    @pl.when(s == NS - 1)
    def _ztail():
        pltpu.sync_copy(zbuf.at[pl.ds(0, 16)], acc.at[pl.ds(NS * SZ, 16)])

    plsc.subcore_barrier()

    wait_idx(0, 0)
    start_gather(0)

    def scale(j, b):
        rb = rbufs[b]
        eb = ewbs[b]

        @plsc.parallel_loop(0, K, 1, unroll=4)
        def _edge(e):
            w = plsc.load_gather(eb, [jnp.full((16,), e, jnp.int32)])
            for g in range(D // 16):
                rb[e, pl.ds(g * 16, 16)] = rb[e, pl.ds(g * 16, 16)] * w

    def chunk_step(j, b, wait_prev, prefetch, next_gather):
        wait_gather(b)
        if next_gather:
            wait_idx(j + 1, (b + 1) % 3)
            start_gather((b + 1) % 3)
        scale(j, b)
        start_scatter(b)
        if wait_prev:
            wait_scatter((b + 2) % 3)
        if prefetch:
            start_idx(j + 2, (b + 2) % 3)

    chunk_step(0, 0, False, True, True)

    def loop_body(j2, _):
        j = 1 + 3 * j2
        chunk_step(j, 1, True, True, True)
        chunk_step(j + 1, 2, True, True, True)
        chunk_step(j + 2, 0, True, True, True)
        return 0

    lax.fori_loop(0, (NCHUNK - 5) // 3, loop_body, 0)

    chunk_step(NCHUNK - 4, 1, True, True, True)
    chunk_step(NCHUNK - 3, 2, True, True, True)
    chunk_step(NCHUNK - 2, 0, True, False, True)
    chunk_step(NCHUNK - 1, 1, True, False, False)
    wait_scatter(1)

    plsc.subcore_barrier()

    def outcopy(m, _):
        lo = s * SZ + m * ZR
        pltpu.sync_copy(acc.at[pl.ds(lo, ZR)], out_ref.at[c, pl.ds(lo, ZR)])
        return 0

    lax.fori_loop(0, SZ // ZR, outcopy, 0)

    @pl.when(s == NS - 1)
    def _otail():
        pltpu.sync_copy(acc.at[pl.ds(NS * SZ, 16)],
                        out_ref.at[c, pl.ds(NS * SZ, 16)])


def _gru(W0, WihT, WhhT, bih, bhh):
    gi = jnp.dot(W0, WihT, preferred_element_type=f32) + bih
    gh = jnp.dot(W0, WhhT, preferred_element_type=f32) + bhh
    r = jax.nn.sigmoid(gi[:, :D] + gh[:, :D])
    z = jax.nn.sigmoid(gi[:, D:2 * D] + gh[:, D:2 * D])
    n = jnp.tanh(gi[:, 2 * D:] + r * gh[:, 2 * D:])
    return (1.0 - z) * n + z * W0


def _tc_gru_body(W01, WihT1, WhhT1, bih1, bhh1, W02, WihT2, WhhT2, bih2, bhh2,
                 W1_out, W2_out):
    W1_out[...] = _gru(W01[...], WihT1[...], WhhT1[...], bih1[...], bhh1[...])
    W2_out[...] = _gru(W02[...], WihT2[...], WhhT2[...], bih2[...], bhh2[...])


def _tc_gru(W01, WihT1, WhhT1, bih1, bhh1, W02, WihT2, WhhT2, bih2, bhh2):
    return pl.pallas_call(
        _tc_gru_body,
        out_shape=(jax.ShapeDtypeStruct((D, D), f32),
                   jax.ShapeDtypeStruct((D, D), f32)),
    )(W01, WihT1, WhhT1, bih1, bhh1, W02, WihT2, WhhT2, bih2, bhh2)


_RB = 1000
_NG = N // _RB


def _tc_xw1_body(x_ref, W1_ref, degT_ref, xw_ref, dinv_ref):
    deg = jnp.sum(degT_ref[...], axis=1, keepdims=True) + 1.0
    dinv = lax.rsqrt(deg)
    dinv_ref[...] = dinv
    xw_ref[...] = dinv * jnp.dot(x_ref[...], W1_ref[...],
                                 preferred_element_type=f32)


def _tc_xw1(x, W1, deg_partsT):
    return pl.pallas_call(
        _tc_xw1_body,
        grid=(_NG,),
        in_specs=[
            pl.BlockSpec((_RB, D), lambda i: (i, 0)),
            pl.BlockSpec((D, D), lambda i: (0, 0)),
            pl.BlockSpec((_RB, NT), lambda i: (i, 0)),
        ],
        out_specs=(
            pl.BlockSpec((_RB, D), lambda i: (i, 0)),
            pl.BlockSpec((_RB, 1), lambda i: (i, 0)),
        ),
        out_shape=(jax.ShapeDtypeStruct((N, D), f32),
                   jax.ShapeDtypeStruct((N, 1), f32)),
    )(x, W1, deg_partsT)


def _tc_mid_body(p_ref, xw_ref, dinv_ref, W2_ref, out_ref):
    dinv = dinv_ref[...]
    h = jnp.maximum(dinv * (p_ref[0] + p_ref[1] + xw_ref[...]), 0.0)
    out_ref[...] = dinv * jnp.dot(h, W2_ref[...], preferred_element_type=f32)


def _tc_mid(p, xw1p, dinv, W2):
    return pl.pallas_call(
        _tc_mid_body,
        grid=(_NG,),
        in_specs=[
            pl.BlockSpec((NC, _RB, D), lambda i: (0, i, 0)),
            pl.BlockSpec((_RB, D), lambda i: (i, 0)),
            pl.BlockSpec((_RB, 1), lambda i: (i, 0)),
            pl.BlockSpec((D, D), lambda i: (0, 0)),
        ],
        out_specs=pl.BlockSpec((_RB, D), lambda i: (i, 0)),
        out_shape=jax.ShapeDtypeStruct((N, D), f32),
    )(p, xw1p, dinv, W2)


def _tc_final_body(q_ref, xw_ref, dinv_ref, WlinT_ref, blin_ref, out_ref):
    dinv = dinv_ref[...]
    h = jnp.maximum(dinv * (q_ref[0] + q_ref[1] + xw_ref[...]), 0.0)
    logits = jnp.dot(h, WlinT_ref[...], preferred_element_type=f32) + blin_ref[...]
    m = jnp.max(logits, axis=-1, keepdims=True)
    lse = m + jnp.log(jnp.sum(jnp.exp(logits - m), axis=-1, keepdims=True))
    out_ref[...] = logits - lse


def _tc_final(q, xw2p, dinv, WlinT, blin2):
    return pl.pallas_call(
        _tc_final_body,
        grid=(_NG,),
        in_specs=[
            pl.BlockSpec((NC, _RB, D), lambda i: (0, i, 0)),
            pl.BlockSpec((_RB, D), lambda i: (i, 0)),
            pl.BlockSpec((_RB, 1), lambda i: (i, 0)),
            pl.BlockSpec((D, C), lambda i: (0, 0)),
            pl.BlockSpec((1, C), lambda i: (0, 0)),
        ],
        out_specs=pl.BlockSpec((_RB, C), lambda i: (i, 0)),
        out_shape=jax.ShapeDtypeStruct((N, C), f32),
    )(q, xw2p, dinv, WlinT, blin2)


def kernel(x, edge_index, edge_weight, W0_1, Wih_1, Whh_1, bih_1, bhh_1,
           W0_2, Wih_2, Whh_2, bih_2, bhh_2, Wlin, blin):
    edge_flat = edge_index.reshape(2 * E)

    deg_parts = _sc_deg(edge_flat, edge_weight)
    W1, W2 = _tc_gru(
        W0_1, Wih_1.T, Whh_1.T, bih_1.reshape(1, 3 * D), bhh_1.reshape(1, 3 * D),
        W0_2, Wih_2.T, Whh_2.T, bih_2.reshape(1, 3 * D), bhh_2.reshape(1, 3 * D),
    )
    xw1p, dinv = _tc_xw1(x, W1, deg_parts.T)
    p = _sc_conv(xw1p, edge_flat, edge_weight)
    xw2p = _tc_mid(p, xw1p, dinv, W2)
    q = _sc_conv(xw2p, edge_flat, edge_weight)
    return _tc_final(q, xw2p, dinv, Wlin.T, blin.reshape(1, C))


# speedup vs baseline: 27.2096x; 1.0538x over previous
import functools

import jax
import jax.numpy as jnp
from jax import lax
from jax.experimental import pallas as pl
from jax.experimental.pallas import tpu as pltpu
from jax.experimental.pallas import tpu_sc as plsc

N = 10000
E = 320000
D = 128
C = 40

NC = 2
NS = 16
NT = NC * NS
EPT = E // NT
K = 80
NCHUNK = EPT // K
SZ = 624
ZR = 104

_mesh = plsc.VectorSubcoreMesh(
    core_axis_name="c", subcore_axis_name="s", num_cores=NC, num_subcores=NS
)
_sc_params = pltpu.CompilerParams(needs_layout_passes=False)

f32 = jnp.float32


@functools.partial(
    pl.kernel,
    out_type=jax.ShapeDtypeStruct((NT, N), f32),
    mesh=_mesh,
    compiler_params=_sc_params,
    scratch_types=[
        pltpu.VMEM((N,), f32),
        pltpu.VMEM((EPT,), jnp.int32),
        pltpu.VMEM((EPT,), f32),
        pltpu.SemaphoreType.DMA,
        pltpu.SemaphoreType.DMA,
    ],
)
def _sc_deg(eflat_ref, ew_ref, out_ref, degl, colb, ewb, semc, semw):
    c = lax.axis_index("c")
    s = lax.axis_index("s")
    wid = c * NS + s

    pltpu.async_copy(eflat_ref.at[pl.ds(E + wid * EPT, EPT)], colb, semc)
    pltpu.async_copy(ew_ref.at[pl.ds(wid * EPT, EPT)], ewb, semw)

    def zero_body(i, _):
        degl[pl.ds(i * 16, 16)] = jnp.zeros((16,), f32)
        return 0

    lax.fori_loop(0, N // 16, zero_body, 0)

    pltpu.make_async_copy(
        eflat_ref.at[pl.ds(E + wid * EPT, EPT)], colb, semc).wait()
    pltpu.make_async_copy(ew_ref.at[pl.ds(wid * EPT, EPT)], ewb, semw).wait()

    def vec_body(i, _):
        cv = colb[pl.ds(i * 16, 16)]
        ev = ewb[pl.ds(i * 16, 16)]
        plsc.addupdate_scatter(degl, [cv], ev)
        return 0

    lax.fori_loop(0, EPT // 16, vec_body, 0)
    pltpu.sync_copy(degl, out_ref.at[wid])


@functools.partial(
    pl.kernel,
    out_type=jax.ShapeDtypeStruct((NC, N, D), f32),
    mesh=_mesh,
    compiler_params=_sc_params,
    scratch_types=[
        pltpu.VMEM_SHARED((N, D), f32),
        pltpu.VMEM((ZR, D), f32),
        pltpu.VMEM((K,), f32),
        pltpu.VMEM((K,), f32),
        pltpu.VMEM((K,), f32),
        pltpu.VMEM((K,), jnp.int32),
        pltpu.VMEM((K,), jnp.int32),
        pltpu.VMEM((K,), jnp.int32),
        pltpu.VMEM((K,), jnp.int32),
        pltpu.VMEM((K,), jnp.int32),
        pltpu.VMEM((K,), jnp.int32),
        pltpu.VMEM((K, D), f32),
        pltpu.VMEM((K, D), f32),
        pltpu.VMEM((K, D), f32),
        pltpu.SemaphoreType.DMA,
        pltpu.SemaphoreType.DMA,
        pltpu.SemaphoreType.DMA,
        pltpu.SemaphoreType.DMA,
        pltpu.SemaphoreType.DMA,
        pltpu.SemaphoreType.DMA,
        pltpu.SemaphoreType.DMA,
        pltpu.SemaphoreType.DMA,
        pltpu.SemaphoreType.DMA,
    ],
)
def _sc_conv(xw_ref, eflat_ref, ew_ref, out_ref, acc, zbuf,
             ewb0, ewb1, ewb2, rowb0, rowb1, rowb2, colb0, colb1, colb2,
             rbuf0, rbuf1, rbuf2, si0, si1, si2, sg0, sg1, sg2,
             ss0, ss1, ss2):
    c = lax.axis_index("c")
    s = lax.axis_index("s")
    wid = c * NS + s
    ewbs = (ewb0, ewb1, ewb2)
    rowbs = (rowb0, rowb1, rowb2)
    colbs = (colb0, colb1, colb2)
    rbufs = (rbuf0, rbuf1, rbuf2)
    sis = (si0, si1, si2)
    sgs = (sg0, sg1, sg2)
    sss = (ss0, ss1, ss2)

    def start_idx(j, b):
        base = wid * EPT + j * K
        pltpu.async_copy(eflat_ref.at[pl.ds(base, K)], rowbs[b], sis[b])
        pltpu.async_copy(eflat_ref.at[pl.ds(E + base, K)], colbs[b], sis[b])
        pltpu.async_copy(ew_ref.at[pl.ds(base, K)], ewbs[b], sis[b])

    def wait_idx(j, b):
        base = wid * EPT + j * K
        pltpu.make_async_copy(
            eflat_ref.at[pl.ds(base, K)], rowbs[b], sis[b]).wait()
        pltpu.make_async_copy(
            eflat_ref.at[pl.ds(E + base, K)], colbs[b], sis[b]).wait()
        pltpu.make_async_copy(
            ew_ref.at[pl.ds(base, K)], ewbs[b], sis[b]).wait()

    def start_gather(b):
        pltpu.async_copy(xw_ref.at[rowbs[b]], rbufs[b], sgs[b])

    def wait_gather(b):
        pltpu.make_async_copy(xw_ref.at[rowbs[b]], rbufs[b], sgs[b]).wait()

    def start_scatter(b):
        pltpu.async_copy(rbufs[b], acc.at[colbs[b]], sss[b], add=True)

    def wait_scatter(b):
        pltpu.make_async_copy(rbufs[b], acc.at[colbs[b]], sss[b]).wait()

    start_idx(0, 0)
    start_idx(1, 1)

    def zrow(r, _):
        for g in range(D // 16):
            zbuf[r, pl.ds(g * 16, 16)] = jnp.zeros((16,), f32)
        return 0

    lax.fori_loop(0, ZR, zrow, 0)

    def zcopy(m, _):
        pltpu.sync_copy(zbuf, acc.at[pl.ds(s * SZ + m * ZR, ZR)])
        return 0

    lax.fori_loop(0, SZ // ZR, zcopy, 0)

    @pl.when(s == NS - 1)
    def _ztail():
        pltpu.sync_copy(zbuf.at[pl.ds(0, 16)], acc.at[pl.ds(NS * SZ, 16)])

    plsc.subcore_barrier()

    wait_idx(0, 0)
    start_gather(0)

    def scale(j, b):
        rb = rbufs[b]
        eb = ewbs[b]

        @plsc.parallel_loop(0, K, 1, unroll=4)
        def _edge(e):
            w = plsc.load_gather(eb, [jnp.full((16,), e, jnp.int32)])
            for g in range(D // 16):
                rb[e, pl.ds(g * 16, 16)] = rb[e, pl.ds(g * 16, 16)] * w

    def chunk_step(j, b, wait_prev, prefetch, next_gather):
        if next_gather:
            wait_idx(j + 1, (b + 1) % 3)
            start_gather((b + 1) % 3)
        wait_gather(b)
        scale(j, b)
        start_scatter(b)
        if wait_prev:
            wait_scatter((b + 2) % 3)
        if prefetch:
            start_idx(j + 2, (b + 2) % 3)

    chunk_step(0, 0, False, True, True)

    def loop_body(j2, _):
        j = 1 + 3 * j2
        chunk_step(j, 1, True, True, True)
        chunk_step(j + 1, 2, True, True, True)
        chunk_step(j + 2, 0, True, True, True)
        return 0

    lax.fori_loop(0, (NCHUNK - 5) // 3, loop_body, 0)

    chunk_step(NCHUNK - 4, 1, True, True, True)
    chunk_step(NCHUNK - 3, 2, True, True, True)
    chunk_step(NCHUNK - 2, 0, True, False, True)
    chunk_step(NCHUNK - 1, 1, True, False, False)
    wait_scatter(1)

    plsc.subcore_barrier()

    def outcopy(m, _):
        lo = s * SZ + m * ZR
        pltpu.sync_copy(acc.at[pl.ds(lo, ZR)], out_ref.at[c, pl.ds(lo, ZR)])
        return 0

    lax.fori_loop(0, SZ // ZR, outcopy, 0)

    @pl.when(s == NS - 1)
    def _otail():
        pltpu.sync_copy(acc.at[pl.ds(NS * SZ, 16)],
                        out_ref.at[c, pl.ds(NS * SZ, 16)])


def _gru(W0, WihT, WhhT, bih, bhh):
    gi = jnp.dot(W0, WihT, preferred_element_type=f32) + bih
    gh = jnp.dot(W0, WhhT, preferred_element_type=f32) + bhh
    r = jax.nn.sigmoid(gi[:, :D] + gh[:, :D])
    z = jax.nn.sigmoid(gi[:, D:2 * D] + gh[:, D:2 * D])
    n = jnp.tanh(gi[:, 2 * D:] + r * gh[:, 2 * D:])
    return (1.0 - z) * n + z * W0


def _tc_gru_body(W01, WihT1, WhhT1, bih1, bhh1, W02, WihT2, WhhT2, bih2, bhh2,
                 W1_out, W2_out):
    W1_out[...] = _gru(W01[...], WihT1[...], WhhT1[...], bih1[...], bhh1[...])
    W2_out[...] = _gru(W02[...], WihT2[...], WhhT2[...], bih2[...], bhh2[...])


def _tc_gru(W01, WihT1, WhhT1, bih1, bhh1, W02, WihT2, WhhT2, bih2, bhh2):
    return pl.pallas_call(
        _tc_gru_body,
        out_shape=(jax.ShapeDtypeStruct((D, D), f32),
                   jax.ShapeDtypeStruct((D, D), f32)),
    )(W01, WihT1, WhhT1, bih1, bhh1, W02, WihT2, WhhT2, bih2, bhh2)


_RB = 1000
_NG = N // _RB


def _tc_xw1_body(x_ref, W1_ref, degT_ref, xw_ref, dinv_ref):
    deg = jnp.sum(degT_ref[...], axis=1, keepdims=True) + 1.0
    dinv = lax.rsqrt(deg)
    dinv_ref[...] = dinv
    xw_ref[...] = dinv * jnp.dot(x_ref[...], W1_ref[...],
                                 preferred_element_type=f32)


def _tc_xw1(x, W1, deg_partsT):
    return pl.pallas_call(
        _tc_xw1_body,
        grid=(_NG,),
        in_specs=[
            pl.BlockSpec((_RB, D), lambda i: (i, 0)),
            pl.BlockSpec((D, D), lambda i: (0, 0)),
            pl.BlockSpec((_RB, NT), lambda i: (i, 0)),
        ],
        out_specs=(
            pl.BlockSpec((_RB, D), lambda i: (i, 0)),
            pl.BlockSpec((_RB, 1), lambda i: (i, 0)),
        ),
        out_shape=(jax.ShapeDtypeStruct((N, D), f32),
                   jax.ShapeDtypeStruct((N, 1), f32)),
    )(x, W1, deg_partsT)


def _tc_mid_body(p_ref, xw_ref, dinv_ref, W2_ref, out_ref):
    dinv = dinv_ref[...]
    h = jnp.maximum(dinv * (p_ref[0] + p_ref[1] + xw_ref[...]), 0.0)
    out_ref[...] = dinv * jnp.dot(h, W2_ref[...], preferred_element_type=f32)


def _tc_mid(p, xw1p, dinv, W2):
    return pl.pallas_call(
        _tc_mid_body,
        grid=(_NG,),
        in_specs=[
            pl.BlockSpec((NC, _RB, D), lambda i: (0, i, 0)),
            pl.BlockSpec((_RB, D), lambda i: (i, 0)),
            pl.BlockSpec((_RB, 1), lambda i: (i, 0)),
            pl.BlockSpec((D, D), lambda i: (0, 0)),
        ],
        out_specs=pl.BlockSpec((_RB, D), lambda i: (i, 0)),
        out_shape=jax.ShapeDtypeStruct((N, D), f32),
    )(p, xw1p, dinv, W2)


def _tc_final_body(q_ref, xw_ref, dinv_ref, WlinT_ref, blin_ref, out_ref):
    dinv = dinv_ref[...]
    h = jnp.maximum(dinv * (q_ref[0] + q_ref[1] + xw_ref[...]), 0.0)
    logits = jnp.dot(h, WlinT_ref[...], preferred_element_type=f32) + blin_ref[...]
    m = jnp.max(logits, axis=-1, keepdims=True)
    lse = m + jnp.log(jnp.sum(jnp.exp(logits - m), axis=-1, keepdims=True))
    out_ref[...] = logits - lse


def _tc_final(q, xw2p, dinv, WlinT, blin2):
    return pl.pallas_call(
        _tc_final_body,
        grid=(_NG,),
        in_specs=[
            pl.BlockSpec((NC, _RB, D), lambda i: (0, i, 0)),
            pl.BlockSpec((_RB, D), lambda i: (i, 0)),
            pl.BlockSpec((_RB, 1), lambda i: (i, 0)),
            pl.BlockSpec((D, C), lambda i: (0, 0)),
            pl.BlockSpec((1, C), lambda i: (0, 0)),
        ],
        out_specs=pl.BlockSpec((_RB, C), lambda i: (i, 0)),
        out_shape=jax.ShapeDtypeStruct((N, C), f32),
    )(q, xw2p, dinv, WlinT, blin2)


def kernel(x, edge_index, edge_weight, W0_1, Wih_1, Whh_1, bih_1, bhh_1,
           W0_2, Wih_2, Whh_2, bih_2, bhh_2, Wlin, blin):
    edge_flat = edge_index.reshape(2 * E)

    deg_parts = _sc_deg(edge_flat, edge_weight)
    W1, W2 = _tc_gru(
        W0_1, Wih_1.T, Whh_1.T, bih_1.reshape(1, 3 * D), bhh_1.reshape(1, 3 * D),
        W0_2, Wih_2.T, Whh_2.T, bih_2.reshape(1, 3 * D), bhh_2.reshape(1, 3 * D),
    )
    xw1p, dinv = _tc_xw1(x, W1, deg_parts.T)
    p = _sc_conv(xw1p, edge_flat, edge_weight)
    xw2p = _tc_mid(p, xw1p, dinv, W2)
    q = _sc_conv(xw2p, edge_flat, edge_weight)
    return _tc_final(q, xw2p, dinv, Wlin.T, blin.reshape(1, C))
